# baseline (speedup 1.0000x reference)
import jax
import jax.numpy as jnp
from jax import lax
from jax.experimental import pallas as pl
from jax.experimental.pallas import tpu as pltpu

D_MODEL = 64
N_ROWS = 8192
N_CODES = 8192
BM = 1024
BN = 1024
_NI = N_ROWS // BM
_NJ = N_CODES // BN
_I32_MAX = 2**31 - 1


def _argmin_body(z_ref, cb_ref, zsq_ref, csq_ref, idx_out, dmin_out):
    zb = z_ref[...]
    zsqb = zsq_ref[...]

    def step(j, carry):
        best_val, best_idx = carry
        off = pl.multiple_of(j * BN, BN)
        cbb = cb_ref[pl.ds(off, BN), :]
        csqb = csq_ref[:, pl.ds(off, BN)]
        mm = lax.dot_general(zb, cbb, (((1,), (1,)), ((), ())),
                             preferred_element_type=jnp.float32)
        d = (zsqb + csqb) - 2.0 * mm
        local_min = jnp.min(d, axis=1, keepdims=True)
        col = lax.broadcasted_iota(jnp.int32, (BM, BN), 1) + j * BN
        local_arg = jnp.min(jnp.where(d == local_min, col, _I32_MAX),
                            axis=1, keepdims=True)
        better = local_min < best_val
        return (jnp.where(better, local_min, best_val),
                jnp.where(better, local_arg, best_idx))

    init = (jnp.full((BM, 1), jnp.inf, jnp.float32),
            jnp.full((BM, 1), _I32_MAX, jnp.int32))
    best_val, best_idx = lax.fori_loop(0, _NJ, step, init)
    idx_out[...] = best_idx
    dmin_out[...] = best_val


def _distance_argmin(flat_z, codebook, zsq, csq, interpret=False):
    return pl.pallas_call(
        _argmin_body,
        grid=(_NI,),
        in_specs=[
            pl.BlockSpec((BM, D_MODEL), lambda i: (i, 0)),
            pl.BlockSpec((N_CODES, D_MODEL), lambda i: (0, 0)),
            pl.BlockSpec((BM, 1), lambda i: (i, 0)),
            pl.BlockSpec((1, N_CODES), lambda i: (0, 0)),
        ],
        out_specs=[
            pl.BlockSpec((BM, 1), lambda i: (i, 0)),
            pl.BlockSpec((BM, 1), lambda i: (i, 0)),
        ],
        out_shape=[
            jax.ShapeDtypeStruct((N_ROWS, 1), jnp.int32),
            jax.ShapeDtypeStruct((N_ROWS, 1), jnp.float32),
        ],
        interpret=interpret,
    )(flat_z, codebook, zsq, csq)


def kernel(z, codebook):
    flat_z = z.reshape(-1, D_MODEL)
    zsq = (flat_z ** 2).sum(axis=-1, keepdims=True)
    csq = (codebook ** 2).sum(axis=-1).reshape(1, N_CODES)
    idx2d, dmin = _distance_argmin(flat_z, codebook, zsq, csq)
    indices = idx2d.reshape(N_ROWS)
    quantized = jnp.take(codebook, indices, axis=0).reshape(z.shape)
    quantized_st = z + (quantized - z)
    loss = (1.25 * jnp.sum(dmin) / (N_ROWS * D_MODEL)).reshape(())
    indices_out = indices.reshape(z.shape[:-1])
    return (quantized_st, loss, indices_out)
